# Initial kernel scaffold; baseline (speedup 1.0000x reference)
import jax
import jax.numpy as jnp
from jax.experimental import pallas as pl


def kernel(feat1, feat2, msk, samp_bias1, samp_bias2, edge_index, W_prep, W_self, W_neigh, W_att, v_att, W_fc, b_fc, W_disc):
    raise NotImplementedError("write your pallas kernel here")


# speedup vs baseline: 3.5258x; 3.5258x over previous
import functools

import jax
import jax.numpy as jnp
from jax import lax
from jax.experimental import pallas as pl
from jax.experimental.pallas import tpu as pltpu
from jax.experimental.pallas import tpu_sc as plsc

NC, NS = 2, 16
NW = NC * NS

_MM = dict(preferred_element_type=jnp.float32,
           precision=lax.Precision.HIGHEST)


def _deg_partials(dst, n):
    e = dst.shape[0]
    per_w = e // NW
    ch = 2000
    assert e % NW == 0 and per_w % ch == 0
    zr = (n // NS) // 8 * 8
    tail = n - zr * NS
    mesh = plsc.VectorSubcoreMesh(core_axis_name="c", subcore_axis_name="s")

    @functools.partial(
        pl.kernel,
        out_type=jax.ShapeDtypeStruct((NC, n, 16), jnp.float32),
        mesh=mesh,
        scratch_types=[
            pltpu.VMEM((ch,), jnp.int32),
            pltpu.VMEM((ch, 16), jnp.float32),
            pltpu.VMEM((zr, 16), jnp.float32),
            pltpu.VMEM_SHARED((n, 16), jnp.float32),
        ],
        compiler_params=pltpu.CompilerParams(use_tc_tiling_on_sc=False),
    )
    def deg_kernel(dst_hbm, out_hbm, didx, ones_v, zbuf, acc):
        c = lax.axis_index("c")
        s = lax.axis_index("s")
        wid = s * NC + c

        def fill_ones(k, _):
            ones_v[k] = jnp.ones((16,), jnp.float32)
            return 0

        lax.fori_loop(0, ch, fill_ones, 0)

        def fill_zero(k, _):
            zbuf[k] = jnp.zeros((16,), jnp.float32)
            return 0

        lax.fori_loop(0, zr, fill_zero, 0)

        lo = pl.multiple_of(s * zr, 8)
        pltpu.sync_copy(zbuf, acc.at[pl.ds(lo, zr)])

        @pl.when(s == NS - 1)
        def _():
            pltpu.sync_copy(zbuf.at[pl.ds(0, tail)],
                            acc.at[pl.ds(n - tail, tail)])

        plsc.subcore_barrier()

        def step(i, _):
            base = wid * per_w + i * ch
            pltpu.sync_copy(dst_hbm.at[pl.ds(base, ch)], didx)
            pltpu.sync_copy(ones_v, acc.at[didx], add=True)
            return 0

        lax.fori_loop(0, per_w // ch, step, 0)
        plsc.subcore_barrier()
        pltpu.sync_copy(acc.at[pl.ds(lo, zr)], zbuf)
        pltpu.sync_copy(zbuf, out_hbm.at[c, pl.ds(lo, zr)])

        @pl.when(s == NS - 1)
        def _():
            pltpu.sync_copy(acc.at[pl.ds(n - tail, tail)],
                            zbuf.at[pl.ds(0, tail)])
            pltpu.sync_copy(zbuf.at[pl.ds(0, tail)],
                            out_hbm.at[c, pl.ds(n - tail, tail)])

    return deg_kernel(dst)


def _segsum_partials(tables, src, dst):
    g_cnt = len(tables)
    n, h = tables[0].shape
    e = src.shape[0]
    per_w = e // NW
    ch = 200
    assert e % NW == 0 and per_w % ch == 0
    zr = (n // NS) // 8 * 8
    tail = n - zr * NS
    chunks = [(o, min(ch, zr - o)) for o in range(0, zr, ch)]
    assert all(sz % 8 == 0 for _, sz in chunks) and tail % 8 == 0
    assert tail <= ch
    mesh = plsc.VectorSubcoreMesh(core_axis_name="c", subcore_axis_name="s")

    @functools.partial(
        pl.kernel,
        out_type=jax.ShapeDtypeStruct((NC, g_cnt, n, h), jnp.float32),
        mesh=mesh,
        scratch_types=[
            pltpu.VMEM((ch,), jnp.int32),
            pltpu.VMEM((ch,), jnp.int32),
            pltpu.VMEM((ch, h), jnp.float32),
            pltpu.SemaphoreType.DMA,
            pltpu.VMEM_SHARED((n, h), jnp.float32),
        ],
    )
    def seg_kernel(*refs):
        tbls = refs[:g_cnt]
        src_hbm = refs[g_cnt]
        dst_hbm = refs[g_cnt + 1]
        out_hbm = refs[g_cnt + 2]
        sidx, didx, rows, sem, acc = refs[g_cnt + 3:]
        c = lax.axis_index("c")
        s = lax.axis_index("s")
        wid = s * NC + c
        lo = pl.multiple_of(s * zr, 8)
        hs = h // 16

        def fill_zero(k, _):
            rows[k // hs, pl.ds((k % hs) * 16, 16)] = jnp.zeros(
                (16,), jnp.float32)
            return 0

        for gi in range(g_cnt):
            lax.fori_loop(0, ch * hs, fill_zero, 0)
            for off, sz in chunks:
                pltpu.sync_copy(rows.at[pl.ds(0, sz)],
                                acc.at[pl.ds(pl.multiple_of(lo + off, 8), sz)])

            @pl.when(s == NS - 1)
            def _():
                pltpu.sync_copy(rows.at[pl.ds(0, tail)],
                                acc.at[pl.ds(n - tail, tail)])

            plsc.subcore_barrier()

            def step(i, _):
                base = wid * per_w + i * ch
                pltpu.sync_copy(src_hbm.at[pl.ds(base, ch)], sidx)
                pltpu.sync_copy(dst_hbm.at[pl.ds(base, ch)], didx)
                pltpu.async_copy(tbls[gi].at[sidx], rows, sem).wait()
                pltpu.sync_copy(rows, acc.at[didx], add=True)
                return 0

            lax.fori_loop(0, per_w // ch, step, 0)
            plsc.subcore_barrier()
            for off, sz in chunks:
                o8 = pl.multiple_of(lo + off, 8)
                pltpu.sync_copy(acc.at[pl.ds(o8, sz)], rows.at[pl.ds(0, sz)])
                pltpu.sync_copy(rows.at[pl.ds(0, sz)],
                                out_hbm.at[c, gi, pl.ds(o8, sz)])

            @pl.when(s == NS - 1)
            def _():
                pltpu.sync_copy(acc.at[pl.ds(n - tail, tail)],
                                rows.at[pl.ds(0, tail)])
                pltpu.sync_copy(rows.at[pl.ds(0, tail)],
                                out_hbm.at[c, gi, pl.ds(n - tail, tail)])

            plsc.subcore_barrier()

    return seg_kernel(*tables, src, dst)


def _prep(feats, w_prep, bn=1000):
    f_cnt, n, d = feats.shape
    h = w_prep.shape[1]
    nb = n // bn

    def body(x_ref, w_ref, o_ref):
        o_ref[0] = jnp.maximum(jnp.dot(x_ref[0], w_ref[...], **_MM), 0.0)

    return pl.pallas_call(
        body,
        grid=(f_cnt, nb),
        in_specs=[
            pl.BlockSpec((1, bn, d), lambda f, i: (f, i, 0)),
            pl.BlockSpec((d, h), lambda f, i: (0, 0)),
        ],
        out_specs=pl.BlockSpec((1, bn, h), lambda f, i: (f, i, 0)),
        out_shape=jax.ShapeDtypeStruct((f_cnt, n, h), jnp.float32),
    )(feats, w_prep)


def _layer(hs, aggp, degp, ws, wn, shared_input, bn=1000):
    g_out = 4
    n, h = hs.shape[1], hs.shape[2]
    nb = n // bn
    gmap = (lambda g: g // 2) if shared_input else (lambda g: g)

    def body(h_ref, p0_ref, p1_ref, d0_ref, d1_ref, ws_ref, wn_ref, o_ref):
        dp = d0_ref[0][:, :1] + d1_ref[0][:, :1]
        rdeg = 1.0 / jnp.maximum(dp, 1.0)
        a = (p0_ref[0, 0] + p1_ref[0, 0]) * rdeg
        o_ref[0] = jnp.maximum(
            jnp.dot(h_ref[0], ws_ref[0], **_MM)
            + jnp.dot(a, wn_ref[0], **_MM), 0.0)

    return pl.pallas_call(
        body,
        grid=(g_out, nb),
        in_specs=[
            pl.BlockSpec((1, bn, h), lambda g, i: (gmap(g), i, 0)),
            pl.BlockSpec((1, 1, bn, h), lambda g, i: (0, gmap(g), i, 0)),
            pl.BlockSpec((1, 1, bn, h), lambda g, i: (1, gmap(g), i, 0)),
            pl.BlockSpec((1, bn, 16), lambda g, i: (0, i, 0)),
            pl.BlockSpec((1, bn, 16), lambda g, i: (1, i, 0)),
            pl.BlockSpec((1, h, h), lambda g, i: (g % 2, 0, 0)),
            pl.BlockSpec((1, h, h), lambda g, i: (g % 2, 0, 0)),
        ],
        out_specs=pl.BlockSpec((1, bn, h), lambda g, i: (g, i, 0)),
        out_shape=jax.ShapeDtypeStruct((g_out, n, h), jnp.float32),
    )(hs, aggp, aggp, degp, degp, ws, wn)


def _attention(h2s, w_att, v_att_col, msk_col, bn=1000):
    n, h = h2s.shape[1], h2s.shape[2]
    nb = n // bn

    def body(a_ref, b_ref, w_ref, v_ref, m_ref, hsel_ref, al_ref, cs_ref):
        f = pl.program_id(0)
        i = pl.program_id(1)
        ha = a_ref[0]
        hb = b_ref[0]
        w = w_ref[...]
        v = v_ref[...]
        sa = jnp.dot(jnp.tanh(jnp.dot(ha, w, **_MM)), v, **_MM)
        sb = jnp.dot(jnp.tanh(jnp.dot(hb, w, **_MM)), v, **_MM)
        m = jnp.maximum(sa, sb)
        ea = jnp.exp(sa - m)
        eb = jnp.exp(sb - m)
        tot = ea + eb
        aa = ea / tot
        ab = eb / tot
        hsel = aa * ha + ab * hb
        hsel_ref[0] = hsel
        al_ref[0] = jnp.concatenate([aa, ab], axis=1)

        @pl.when(jnp.logical_and(f == 0, i == 0))
        def _():
            cs_ref[...] = jnp.zeros((2, h), jnp.float32)

        @pl.when(f == 0)
        def _():
            mc = m_ref[...]
            cs_ref[0:1, :] += jnp.sum(hsel * mc, axis=0, keepdims=True)
            cs_ref[1:2, :] += jnp.full((1, h), jnp.sum(mc), jnp.float32)

    return pl.pallas_call(
        body,
        grid=(2, nb),
        in_specs=[
            pl.BlockSpec((1, bn, h), lambda f, i: (2 * f, i, 0)),
            pl.BlockSpec((1, bn, h), lambda f, i: (2 * f + 1, i, 0)),
            pl.BlockSpec((h, 64), lambda f, i: (0, 0)),
            pl.BlockSpec((64, 1), lambda f, i: (0, 0)),
            pl.BlockSpec((bn, 1), lambda f, i: (i, 0)),
        ],
        out_specs=[
            pl.BlockSpec((1, bn, h), lambda f, i: (f, i, 0)),
            pl.BlockSpec((1, bn, 2), lambda f, i: (f, i, 0)),
            pl.BlockSpec((2, h), lambda f, i: (0, 0)),
        ],
        out_shape=[
            jax.ShapeDtypeStruct((2, n, h), jnp.float32),
            jax.ShapeDtypeStruct((2, n, 2), jnp.float32),
            jax.ShapeDtypeStruct((2, h), jnp.float32),
        ],
    )(h2s, h2s, w_att, v_att_col, msk_col)


def _head(hsel, colsum, w_disc, w_fc, b_fc_row, sb1_col, sb2_col, bn=1000):
    n, h = hsel.shape[1], hsel.shape[2]
    ncls = w_fc.shape[1]
    nb = n // bn

    def body(h1_ref, h2_ref, cs_ref, wd_ref, wf_ref, bf_ref, s1_ref, s2_ref,
             preds_ref, o1_ref, o2_ref):
        cs = cs_ref[...]
        cvec = jax.nn.sigmoid(cs[0:1, :] / cs[1:2, :])
        t1 = jnp.dot(h1_ref[0], wd_ref[...], **_MM)
        t2 = jnp.dot(h2_ref[0], wd_ref[...], **_MM)
        o1_ref[...] = jnp.sum(t1 * cvec, axis=1, keepdims=True) + s1_ref[...]
        o2_ref[...] = jnp.sum(t2 * cvec, axis=1, keepdims=True) + s2_ref[...]
        preds_ref[...] = jnp.dot(h1_ref[0], wf_ref[...], **_MM) + bf_ref[...]

    return pl.pallas_call(
        body,
        grid=(nb,),
        in_specs=[
            pl.BlockSpec((1, bn, h), lambda i: (0, i, 0)),
            pl.BlockSpec((1, bn, h), lambda i: (1, i, 0)),
            pl.BlockSpec((2, h), lambda i: (0, 0)),
            pl.BlockSpec((h, h), lambda i: (0, 0)),
            pl.BlockSpec((h, ncls), lambda i: (0, 0)),
            pl.BlockSpec((1, ncls), lambda i: (0, 0)),
            pl.BlockSpec((bn, 1), lambda i: (i, 0)),
            pl.BlockSpec((bn, 1), lambda i: (i, 0)),
        ],
        out_specs=[
            pl.BlockSpec((bn, ncls), lambda i: (i, 0)),
            pl.BlockSpec((bn, 1), lambda i: (i, 0)),
            pl.BlockSpec((bn, 1), lambda i: (i, 0)),
        ],
        out_shape=[
            jax.ShapeDtypeStruct((n, ncls), jnp.float32),
            jax.ShapeDtypeStruct((n, 1), jnp.float32),
            jax.ShapeDtypeStruct((n, 1), jnp.float32),
        ],
    )(hsel, hsel, colsum, w_disc, w_fc, b_fc_row, sb1_col, sb2_col)


def kernel(feat1, feat2, msk, samp_bias1, samp_bias2, edge_index, W_prep,
           W_self, W_neigh, W_att, v_att, W_fc, b_fc, W_disc):
    n, d = feat1.shape
    h = W_prep.shape[1]
    src = edge_index[0]
    dst = edge_index[1]

    degp = _deg_partials(dst, n)
    feats = jnp.stack([feat1, feat2])
    h0s = _prep(feats, W_prep)
    a0 = _segsum_partials([h0s[0], h0s[1]], src, dst)
    h1s = _layer(h0s, a0, degp, W_self[:, 0], W_neigh[:, 0],
                 shared_input=True)
    a1 = _segsum_partials([h1s[0], h1s[1], h1s[2], h1s[3]], src, dst)
    h2s = _layer(h1s, a1, degp, W_self[:, 1], W_neigh[:, 1],
                 shared_input=False)
    hsel, alphas, colsum = _attention(h2s, W_att, v_att.reshape(64, 1),
                                      msk.reshape(n, 1))
    preds, sc1, sc2 = _head(hsel, colsum, W_disc, W_fc, b_fc.reshape(1, -1),
                            samp_bias1.reshape(n, 1), samp_bias2.reshape(n, 1))
    reg = jnp.concatenate([sc1.reshape(1, n), sc2.reshape(1, n)], axis=1)
    return preds, alphas[0], reg


# speedup vs baseline: 4.7513x; 1.3476x over previous
import functools

import jax
import jax.numpy as jnp
from jax import lax
from jax.experimental import pallas as pl
from jax.experimental.pallas import tpu as pltpu
from jax.experimental.pallas import tpu_sc as plsc

NC, NS = 2, 16
NW = NC * NS

_MM = dict(preferred_element_type=jnp.float32,
           precision=lax.Precision.HIGHEST)


def _deg_partials(dst, n):
    e = dst.shape[0]
    per_w = e // NW
    ch = 2000
    assert e % NW == 0 and per_w % ch == 0
    zr = (n // NS) // 8 * 8
    tail = n - zr * NS
    mesh = plsc.VectorSubcoreMesh(core_axis_name="c", subcore_axis_name="s")

    @functools.partial(
        pl.kernel,
        out_type=jax.ShapeDtypeStruct((NC, n, 16), jnp.float32),
        mesh=mesh,
        scratch_types=[
            pltpu.VMEM((ch,), jnp.int32),
            pltpu.VMEM((ch, 16), jnp.float32),
            pltpu.VMEM((zr, 16), jnp.float32),
            pltpu.VMEM_SHARED((n, 16), jnp.float32),
        ],
        compiler_params=pltpu.CompilerParams(use_tc_tiling_on_sc=False),
    )
    def deg_kernel(dst_hbm, out_hbm, didx, ones_v, zbuf, acc):
        c = lax.axis_index("c")
        s = lax.axis_index("s")
        wid = s * NC + c

        def fill_ones(k, _):
            ones_v[k] = jnp.ones((16,), jnp.float32)
            return 0

        lax.fori_loop(0, ch, fill_ones, 0)

        def fill_zero(k, _):
            zbuf[k] = jnp.zeros((16,), jnp.float32)
            return 0

        lax.fori_loop(0, zr, fill_zero, 0)

        lo = pl.multiple_of(s * zr, 8)
        pltpu.sync_copy(zbuf, acc.at[pl.ds(lo, zr)])

        @pl.when(s == NS - 1)
        def _():
            pltpu.sync_copy(zbuf.at[pl.ds(0, tail)],
                            acc.at[pl.ds(n - tail, tail)])

        plsc.subcore_barrier()

        def step(i, _):
            base = wid * per_w + i * ch
            pltpu.sync_copy(dst_hbm.at[pl.ds(base, ch)], didx)
            pltpu.sync_copy(ones_v, acc.at[didx], add=True)
            return 0

        lax.fori_loop(0, per_w // ch, step, 0)
        plsc.subcore_barrier()
        pltpu.sync_copy(acc.at[pl.ds(lo, zr)], zbuf)
        pltpu.sync_copy(zbuf, out_hbm.at[c, pl.ds(lo, zr)])

        @pl.when(s == NS - 1)
        def _():
            pltpu.sync_copy(acc.at[pl.ds(n - tail, tail)],
                            zbuf.at[pl.ds(0, tail)])
            pltpu.sync_copy(zbuf.at[pl.ds(0, tail)],
                            out_hbm.at[c, pl.ds(n - tail, tail)])

    return deg_kernel(dst)


def _segsum_partials(tables, src, dst):
    g_cnt = len(tables)
    n, h = tables[0].shape
    e = src.shape[0]
    per_w = e // NW
    ch = 192
    nfull = per_w // ch
    etail = per_w - nfull * ch
    assert e % NW == 0 and nfull % 2 == 0 and etail % 8 == 0 and etail <= ch
    zr = (n // NS) // 8 * 8
    tail = n - zr * NS
    chunks = [(o, min(ch, zr - o)) for o in range(0, zr, ch)]
    assert all(sz % 8 == 0 for _, sz in chunks) and tail % 8 == 0
    assert tail <= ch
    mesh = plsc.VectorSubcoreMesh(core_axis_name="c", subcore_axis_name="s")

    @functools.partial(
        pl.kernel,
        out_type=jax.ShapeDtypeStruct((NC, g_cnt, n, h), jnp.float32),
        mesh=mesh,
        scratch_types=[
            pltpu.VMEM((ch,), jnp.int32),
            pltpu.VMEM((ch,), jnp.int32),
            pltpu.VMEM((ch,), jnp.int32),
            pltpu.VMEM((ch,), jnp.int32),
            pltpu.VMEM((etail,), jnp.int32),
            pltpu.VMEM((etail,), jnp.int32),
            pltpu.VMEM((ch, h), jnp.float32),
            pltpu.VMEM((ch, h), jnp.float32),
            pltpu.SemaphoreType.DMA,
            pltpu.SemaphoreType.DMA,
            pltpu.VMEM_SHARED((n, h), jnp.float32),
        ],
    )
    def seg_kernel(*refs):
        tbls = refs[:g_cnt]
        (src_hbm, dst_hbm, out_hbm, sidx0, sidx1, didx0, didx1, tsidx,
         tdidx, rows0, rows1, sem0, sem1, acc) = refs[g_cnt:]
        sidx = (sidx0, sidx1)
        didx = (didx0, didx1)
        rows = (rows0, rows1)
        sem = (sem0, sem1)
        c = lax.axis_index("c")
        s = lax.axis_index("s")
        wid = s * NC + c
        wbase = wid * per_w
        lo = pl.multiple_of(s * zr, 8)
        hs = h // 16
        n2 = nfull // 2

        def fill_zero(k, _):
            rows0[k // hs, pl.ds((k % hs) * 16, 16)] = jnp.zeros(
                (16,), jnp.float32)
            return 0

        for gi in range(g_cnt):
            lax.fori_loop(0, ch * hs, fill_zero, 0)
            for off, sz in chunks:
                pltpu.sync_copy(rows0.at[pl.ds(0, sz)],
                                acc.at[pl.ds(pl.multiple_of(lo + off, 8), sz)])

            @pl.when(s == NS - 1)
            def _():
                pltpu.sync_copy(rows0.at[pl.ds(0, tail)],
                                acc.at[pl.ds(n - tail, tail)])

            plsc.subcore_barrier()

            for b in range(2):
                base = wbase + b * ch
                pltpu.sync_copy(src_hbm.at[pl.ds(base, ch)], sidx[b])
                pltpu.sync_copy(dst_hbm.at[pl.ds(base, ch)], didx[b])
                pltpu.async_copy(tbls[gi].at[sidx[b]], rows[b], sem[b])

            def pair(j, _):
                for b in range(2):
                    pltpu.make_async_copy(tbls[gi].at[pl.ds(0, ch)],
                                          rows[b], sem[b]).wait()
                    pltpu.sync_copy(rows[b], acc.at[didx[b]], add=True)

                    @pl.when(j < n2 - 1)
                    def _():
                        base = wbase + (2 * j + b + 2) * ch
                        pltpu.sync_copy(src_hbm.at[pl.ds(base, ch)], sidx[b])
                        pltpu.sync_copy(dst_hbm.at[pl.ds(base, ch)], didx[b])
                        pltpu.async_copy(tbls[gi].at[sidx[b]], rows[b], sem[b])
                return 0

            lax.fori_loop(0, n2, pair, 0)
            if etail:
                base = wbase + nfull * ch
                pltpu.sync_copy(src_hbm.at[pl.ds(base, etail)], tsidx)
                pltpu.sync_copy(dst_hbm.at[pl.ds(base, etail)], tdidx)
                pltpu.async_copy(tbls[gi].at[tsidx],
                                 rows0.at[pl.ds(0, etail)], sem0).wait()
                pltpu.sync_copy(rows0.at[pl.ds(0, etail)],
                                acc.at[tdidx], add=True)
            plsc.subcore_barrier()
            for off, sz in chunks:
                o8 = pl.multiple_of(lo + off, 8)
                pltpu.sync_copy(acc.at[pl.ds(o8, sz)], rows0.at[pl.ds(0, sz)])
                pltpu.sync_copy(rows0.at[pl.ds(0, sz)],
                                out_hbm.at[c, gi, pl.ds(o8, sz)])

            @pl.when(s == NS - 1)
            def _():
                pltpu.sync_copy(acc.at[pl.ds(n - tail, tail)],
                                rows0.at[pl.ds(0, tail)])
                pltpu.sync_copy(rows0.at[pl.ds(0, tail)],
                                out_hbm.at[c, gi, pl.ds(n - tail, tail)])

            plsc.subcore_barrier()

    return seg_kernel(*tables, src, dst)


def _prep(feats, w_prep, bn=1000):
    f_cnt, n, d = feats.shape
    h = w_prep.shape[1]
    nb = n // bn

    def body(x_ref, w_ref, o_ref):
        o_ref[0] = jnp.maximum(jnp.dot(x_ref[0], w_ref[...], **_MM), 0.0)

    return pl.pallas_call(
        body,
        grid=(f_cnt, nb),
        in_specs=[
            pl.BlockSpec((1, bn, d), lambda f, i: (f, i, 0)),
            pl.BlockSpec((d, h), lambda f, i: (0, 0)),
        ],
        out_specs=pl.BlockSpec((1, bn, h), lambda f, i: (f, i, 0)),
        out_shape=jax.ShapeDtypeStruct((f_cnt, n, h), jnp.float32),
    )(feats, w_prep)


def _layer(hs, aggp, degp, ws, wn, shared_input, bn=1000):
    g_out = 4
    n, h = hs.shape[1], hs.shape[2]
    nb = n // bn
    gmap = (lambda g: g // 2) if shared_input else (lambda g: g)

    def body(h_ref, p0_ref, p1_ref, d0_ref, d1_ref, ws_ref, wn_ref, o_ref):
        dp = d0_ref[0][:, :1] + d1_ref[0][:, :1]
        rdeg = 1.0 / jnp.maximum(dp, 1.0)
        a = (p0_ref[0, 0] + p1_ref[0, 0]) * rdeg
        o_ref[0] = jnp.maximum(
            jnp.dot(h_ref[0], ws_ref[0], **_MM)
            + jnp.dot(a, wn_ref[0], **_MM), 0.0)

    return pl.pallas_call(
        body,
        grid=(g_out, nb),
        in_specs=[
            pl.BlockSpec((1, bn, h), lambda g, i: (gmap(g), i, 0)),
            pl.BlockSpec((1, 1, bn, h), lambda g, i: (0, gmap(g), i, 0)),
            pl.BlockSpec((1, 1, bn, h), lambda g, i: (1, gmap(g), i, 0)),
            pl.BlockSpec((1, bn, 16), lambda g, i: (0, i, 0)),
            pl.BlockSpec((1, bn, 16), lambda g, i: (1, i, 0)),
            pl.BlockSpec((1, h, h), lambda g, i: (g % 2, 0, 0)),
            pl.BlockSpec((1, h, h), lambda g, i: (g % 2, 0, 0)),
        ],
        out_specs=pl.BlockSpec((1, bn, h), lambda g, i: (g, i, 0)),
        out_shape=jax.ShapeDtypeStruct((g_out, n, h), jnp.float32),
    )(hs, aggp, aggp, degp, degp, ws, wn)


def _attention(h2s, w_att, v_att_col, msk_col, bn=1000):
    n, h = h2s.shape[1], h2s.shape[2]
    nb = n // bn

    def body(a_ref, b_ref, w_ref, v_ref, m_ref, hsel_ref, al_ref, cs_ref):
        f = pl.program_id(0)
        i = pl.program_id(1)
        ha = a_ref[0]
        hb = b_ref[0]
        w = w_ref[...]
        v = v_ref[...]
        sa = jnp.dot(jnp.tanh(jnp.dot(ha, w, **_MM)), v, **_MM)
        sb = jnp.dot(jnp.tanh(jnp.dot(hb, w, **_MM)), v, **_MM)
        m = jnp.maximum(sa, sb)
        ea = jnp.exp(sa - m)
        eb = jnp.exp(sb - m)
        tot = ea + eb
        aa = ea / tot
        ab = eb / tot
        hsel = aa * ha + ab * hb
        hsel_ref[0] = hsel
        al_ref[0] = jnp.concatenate([aa, ab], axis=1)

        @pl.when(jnp.logical_and(f == 0, i == 0))
        def _():
            cs_ref[...] = jnp.zeros((2, h), jnp.float32)

        @pl.when(f == 0)
        def _():
            mc = m_ref[...]
            cs_ref[0:1, :] += jnp.sum(hsel * mc, axis=0, keepdims=True)
            cs_ref[1:2, :] += jnp.full((1, h), jnp.sum(mc), jnp.float32)

    return pl.pallas_call(
        body,
        grid=(2, nb),
        in_specs=[
            pl.BlockSpec((1, bn, h), lambda f, i: (2 * f, i, 0)),
            pl.BlockSpec((1, bn, h), lambda f, i: (2 * f + 1, i, 0)),
            pl.BlockSpec((h, 64), lambda f, i: (0, 0)),
            pl.BlockSpec((64, 1), lambda f, i: (0, 0)),
            pl.BlockSpec((bn, 1), lambda f, i: (i, 0)),
        ],
        out_specs=[
            pl.BlockSpec((1, bn, h), lambda f, i: (f, i, 0)),
            pl.BlockSpec((1, bn, 2), lambda f, i: (f, i, 0)),
            pl.BlockSpec((2, h), lambda f, i: (0, 0)),
        ],
        out_shape=[
            jax.ShapeDtypeStruct((2, n, h), jnp.float32),
            jax.ShapeDtypeStruct((2, n, 2), jnp.float32),
            jax.ShapeDtypeStruct((2, h), jnp.float32),
        ],
    )(h2s, h2s, w_att, v_att_col, msk_col)


def _head(hsel, colsum, w_disc, w_fc, b_fc_row, sb1_col, sb2_col, bn=1000):
    n, h = hsel.shape[1], hsel.shape[2]
    ncls = w_fc.shape[1]
    nb = n // bn

    def body(h1_ref, h2_ref, cs_ref, wd_ref, wf_ref, bf_ref, s1_ref, s2_ref,
             preds_ref, o1_ref, o2_ref):
        cs = cs_ref[...]
        cvec = jax.nn.sigmoid(cs[0:1, :] / cs[1:2, :])
        t1 = jnp.dot(h1_ref[0], wd_ref[...], **_MM)
        t2 = jnp.dot(h2_ref[0], wd_ref[...], **_MM)
        o1_ref[...] = jnp.sum(t1 * cvec, axis=1, keepdims=True) + s1_ref[...]
        o2_ref[...] = jnp.sum(t2 * cvec, axis=1, keepdims=True) + s2_ref[...]
        preds_ref[...] = jnp.dot(h1_ref[0], wf_ref[...], **_MM) + bf_ref[...]

    return pl.pallas_call(
        body,
        grid=(nb,),
        in_specs=[
            pl.BlockSpec((1, bn, h), lambda i: (0, i, 0)),
            pl.BlockSpec((1, bn, h), lambda i: (1, i, 0)),
            pl.BlockSpec((2, h), lambda i: (0, 0)),
            pl.BlockSpec((h, h), lambda i: (0, 0)),
            pl.BlockSpec((h, ncls), lambda i: (0, 0)),
            pl.BlockSpec((1, ncls), lambda i: (0, 0)),
            pl.BlockSpec((bn, 1), lambda i: (i, 0)),
            pl.BlockSpec((bn, 1), lambda i: (i, 0)),
        ],
        out_specs=[
            pl.BlockSpec((bn, ncls), lambda i: (i, 0)),
            pl.BlockSpec((bn, 1), lambda i: (i, 0)),
            pl.BlockSpec((bn, 1), lambda i: (i, 0)),
        ],
        out_shape=[
            jax.ShapeDtypeStruct((n, ncls), jnp.float32),
            jax.ShapeDtypeStruct((n, 1), jnp.float32),
            jax.ShapeDtypeStruct((n, 1), jnp.float32),
        ],
    )(hsel, hsel, colsum, w_disc, w_fc, b_fc_row, sb1_col, sb2_col)


def kernel(feat1, feat2, msk, samp_bias1, samp_bias2, edge_index, W_prep,
           W_self, W_neigh, W_att, v_att, W_fc, b_fc, W_disc):
    n, d = feat1.shape
    h = W_prep.shape[1]
    src = edge_index[0]
    dst = edge_index[1]

    degp = _deg_partials(dst, n)
    feats = jnp.stack([feat1, feat2])
    h0s = _prep(feats, W_prep)
    a0 = _segsum_partials(h0s, src, dst)
    h1s = _layer(h0s, a0, degp, W_self[:, 0], W_neigh[:, 0],
                 shared_input=True)
    a1 = _segsum_partials(h1s, src, dst)
    h2s = _layer(h1s, a1, degp, W_self[:, 1], W_neigh[:, 1],
                 shared_input=False)
    hsel, alphas, colsum = _attention(h2s, W_att, v_att.reshape(64, 1),
                                      msk.reshape(n, 1))
    preds, sc1, sc2 = _head(hsel, colsum, W_disc, W_fc, b_fc.reshape(1, -1),
                            samp_bias1.reshape(n, 1), samp_bias2.reshape(n, 1))
    reg = jnp.concatenate([sc1.reshape(1, n), sc2.reshape(1, n)], axis=1)
    return preds, alphas[0], reg
